# speedup vs baseline: 6.7913x; 6.7913x over previous
import functools

import jax
import jax.numpy as jnp
from jax import lax
from jax.experimental import pallas as pl
from jax.experimental.pallas import tpu as pltpu
from jax.experimental.pallas import tpu_sc as plsc

_G = 128
_K = 2
_NC = 2
_NS = 16
_NW = _NC * _NS


def _make_embed(n_groups: int, dim: int):
    groups_per_w = n_groups // _NW
    n_chunks = groups_per_w // _K
    mesh = plsc.VectorSubcoreMesh(core_axis_name="c", subcore_axis_name="s")

    @functools.partial(
        pl.kernel,
        mesh=mesh,
        out_type=jax.ShapeDtypeStruct((n_groups, _G, dim), jnp.float32),
        scratch_types=[
            pltpu.VMEM((_K, _G), jnp.int32),
            pltpu.VMEM((_K, _G, dim), jnp.float32),
            pltpu.SemaphoreType.DMA,
        ],
    )
    def embed(idx_hbm, table_hbm, out_hbm, idx_v, rows_v, sem):
        wid = lax.axis_index("s") * _NC + lax.axis_index("c")
        base_g = wid * groups_per_w

        def chunk(ci, carry):
            g0 = base_g + ci * _K
            pltpu.sync_copy(idx_hbm.at[pl.ds(g0, _K)], idx_v)
            cps = [
                pltpu.async_copy(table_hbm.at[idx_v.at[j]], rows_v.at[j], sem)
                for j in range(_K)
            ]
            for cp in cps:
                cp.wait()
            pltpu.sync_copy(rows_v, out_hbm.at[pl.ds(g0, _K)])
            return carry

        lax.fori_loop(0, n_chunks, chunk, 0)

    return embed


def kernel(hidden_state, weight):
    b, t = hidden_state.shape
    vocab, dim = weight.shape
    total = b * t
    n_groups = total // _G
    assert total % (_G * _K * _NW) == 0
    idx = hidden_state.reshape(n_groups, _G).astype(jnp.int32)
    embed = _make_embed(n_groups, dim)
    out = embed(idx, weight)
    return out.reshape(b, t, dim)


# speedup vs baseline: 9.1644x; 1.3494x over previous
import functools

import jax
import jax.numpy as jnp
from jax import lax
from jax.experimental import pallas as pl
from jax.experimental.pallas import tpu as pltpu
from jax.experimental.pallas import tpu_sc as plsc

_G = 128
_K = 2
_NC = 2
_NS = 16
_NW = _NC * _NS


def _make_embed(n_groups: int, dim: int):
    gpw = n_groups // _NW
    n_chunks = gpw // _K
    assert n_chunks % 2 == 0 and n_chunks >= 4
    mesh = plsc.VectorSubcoreMesh(core_axis_name="c", subcore_axis_name="s")

    @functools.partial(
        pl.kernel,
        mesh=mesh,
        out_type=jax.ShapeDtypeStruct((n_groups, _G, dim), jnp.float32),
        scratch_types=[
            pltpu.VMEM((gpw, _G), jnp.int32),
            pltpu.VMEM((2, _K, _G, dim), jnp.float32),
            pltpu.SemaphoreType.DMA,
            pltpu.SemaphoreType.DMA,
            pltpu.SemaphoreType.DMA,
            pltpu.SemaphoreType.DMA,
        ],
    )
    def embed(idx_hbm, table_hbm, out_hbm, idx_v, rows_v, sg0, sg1, so0, so1):
        sems_g = (sg0, sg1)
        sems_o = (so0, so1)
        wid = lax.axis_index("s") * _NC + lax.axis_index("c")
        base_g = wid * gpw
        pltpu.sync_copy(idx_hbm.at[pl.ds(base_g, gpw)], idx_v)

        def fire(ci, b):
            for j in range(_K):
                pltpu.async_copy(
                    table_hbm.at[idx_v.at[ci * _K + j]],
                    rows_v.at[b].at[j],
                    sems_g[b],
                )

        def drain_g(b):
            pltpu.make_async_copy(
                out_hbm.at[pl.ds(0, _K)], rows_v.at[b], sems_g[b]
            ).wait()

        def start_out(ci, b):
            pltpu.async_copy(
                rows_v.at[b], out_hbm.at[pl.ds(base_g + ci * _K, _K)], sems_o[b]
            )

        def drain_o(b):
            pltpu.make_async_copy(
                rows_v.at[b], out_hbm.at[pl.ds(0, _K)], sems_o[b]
            ).wait()

        fire(0, 0)
        fire(1, 1)
        drain_g(0)
        start_out(0, 0)

        def step(s, carry):
            for b in range(2):
                i = 1 + 2 * s + b
                bb = (1 + b) % 2
                drain_o(b)
                fire(i + 1, b)
                drain_g(bb)
                start_out(i, bb)
            return carry

        lax.fori_loop(0, (n_chunks - 2) // 2, step, 0)

        drain_o(0)
        drain_g(1)
        start_out(n_chunks - 1, 1)
        drain_o(1)

    return embed


def kernel(hidden_state, weight):
    b, t = hidden_state.shape
    vocab, dim = weight.shape
    total = b * t
    n_groups = total // _G
    assert total % (_G * _K * _NW) == 0
    idx = hidden_state.reshape(n_groups, _G).astype(jnp.int32)
    embed = _make_embed(n_groups, dim)
    out = embed(idx, weight)
    return out.reshape(b, t, dim)


# speedup vs baseline: 9.1866x; 1.0024x over previous
import functools

import jax
import jax.numpy as jnp
from jax import lax
from jax.experimental import pallas as pl
from jax.experimental.pallas import tpu as pltpu
from jax.experimental.pallas import tpu_sc as plsc

_G = 128
_K = 2
_NC = 2
_NS = 16
_NW = _NC * _NS


def _make_embed(n_groups: int, dim: int):
    gpw = n_groups // _NW
    n_chunks = gpw // _K
    assert (n_chunks - 4) % 3 == 0 and n_chunks >= 7
    mesh = plsc.VectorSubcoreMesh(core_axis_name="c", subcore_axis_name="s")

    @functools.partial(
        pl.kernel,
        mesh=mesh,
        out_type=jax.ShapeDtypeStruct((n_groups, _G, dim), jnp.float32),
        scratch_types=[
            pltpu.VMEM((gpw, _G), jnp.int32),
            pltpu.VMEM((3, _K, _G, dim), jnp.float32),
            pltpu.SemaphoreType.DMA,
            pltpu.SemaphoreType.DMA,
            pltpu.SemaphoreType.DMA,
            pltpu.SemaphoreType.DMA,
            pltpu.SemaphoreType.DMA,
            pltpu.SemaphoreType.DMA,
        ],
    )
    def embed(idx_hbm, table_hbm, out_hbm, idx_v, rows_v,
              sg0, sg1, sg2, so0, so1, so2):
        sems_g = (sg0, sg1, sg2)
        sems_o = (so0, so1, so2)
        wid = lax.axis_index("s") * _NC + lax.axis_index("c")
        base_g = wid * gpw
        pltpu.sync_copy(idx_hbm.at[pl.ds(base_g, gpw)], idx_v)

        def fire(ci, b):
            for j in range(_K):
                pltpu.async_copy(
                    table_hbm.at[idx_v.at[ci * _K + j]],
                    rows_v.at[b].at[j],
                    sems_g[b],
                )

        def drain_g(b):
            pltpu.make_async_copy(
                out_hbm.at[pl.ds(0, _K)], rows_v.at[b], sems_g[b]
            ).wait()

        def start_out(ci, b):
            pltpu.async_copy(
                rows_v.at[b], out_hbm.at[pl.ds(base_g + ci * _K, _K)], sems_o[b]
            )

        def drain_o(b):
            pltpu.make_async_copy(
                rows_v.at[b], out_hbm.at[pl.ds(0, _K)], sems_o[b]
            ).wait()

        fire(0, 0)
        fire(1, 1)
        fire(2, 2)
        drain_g(0)
        start_out(0, 0)
        drain_o(0)
        fire(3, 0)
        drain_g(1)
        start_out(1, 1)

        def step(s, carry):
            for b in range(3):
                i = 2 + 3 * s + b
                bb = (2 + b) % 3
                bp = (1 + b) % 3
                drain_o(bp)
                fire(i + 2, bp)
                drain_g(bb)
                start_out(i, bb)
            return carry

        lax.fori_loop(0, (n_chunks - 4) // 3, step, 0)

        drain_o(1)
        drain_g(2)
        start_out(n_chunks - 2, 2)
        drain_o(2)
        drain_g(0)
        start_out(n_chunks - 1, 0)
        drain_o(0)

    return embed


def kernel(hidden_state, weight):
    b, t = hidden_state.shape
    vocab, dim = weight.shape
    total = b * t
    n_groups = total // _G
    assert total % (_G * _K * _NW) == 0
    idx = hidden_state.reshape(n_groups, _G).astype(jnp.int32)
    embed = _make_embed(n_groups, dim)
    out = embed(idx, weight)
    return out.reshape(b, t, dim)
